# baseline (speedup 1.0000x reference)
import functools

import jax
import jax.numpy as jnp
from jax import lax
from jax.experimental import pallas as pl
from jax.experimental.pallas import tpu as pltpu
from jax.experimental.pallas import tpu_sc as plsc

BATCH = 4096
EXP_DIM = 8192
NSEL = 32
CHUNK = 256
POOL = 1024.0
A_MULT = 2654435761.0
M_MOD = 1000000007.0

NC, NS, L = 2, 16, 16
NW = NC * NS
ROWS_PER_W = BATCH // NW
RBLK = 4
NBLK = ROWS_PER_W // RBLK


def _sc_body(x_hbm, out_hbm, buf0, buf1, outb, sem0, sem1):
    cid = lax.axis_index("c")
    sid = lax.axis_index("s")
    wid = sid * NC + cid
    row0 = wid * ROWS_PER_W

    lane = lax.iota(jnp.int32, L)
    wvs = []
    for k in range(16):
        pos = (lane + k * L).astype(jnp.float32)
        wvs.append(jnp.mod(pos * A_MULT, M_MOD))

    bufs = (buf0, buf1)
    sems = (sem0, sem1)

    def dma_in(g, b):
        return pltpu.async_copy(
            x_hbm.at[pl.ds(row0 + g * RBLK, RBLK)], bufs[b], sems[b])

    dma_in(0, 0)
    dma_in(1, 1)

    def compute_block(g, b):
        buf = bufs[b]

        def row_body(r, _):
            def chunk_body(c, carry):
                lo, hi = carry

                def csum(coff):
                    ps = [buf[r, pl.ds(coff + k * L, L)] * wvs[k]
                          for k in range(16)]
                    while len(ps) > 1:
                        ps = [a + b for a, b in zip(ps[::2], ps[1::2])]
                    return jnp.sum(ps[0])

                s_lo = csum(c * CHUNK)
                s_hi = csum(c * CHUNK + 16 * CHUNK)
                lo = jnp.where(lane == c, s_lo, lo)
                hi = jnp.where(lane == c, s_hi, hi)
                return lo, hi

            z = jnp.zeros((L,), jnp.float32)
            lo, hi = lax.fori_loop(0, 16, chunk_body, (z, z))
            row_l = g * RBLK + r
            outb[row_l, pl.ds(0, L)] = jnp.mod(lo, POOL).astype(jnp.int32)
            outb[row_l, pl.ds(L, L)] = jnp.mod(hi, POOL).astype(jnp.int32)
            return 0

        lax.fori_loop(0, RBLK, row_body, 0)

    def pair_body(p, _):
        for b in range(2):
            g = 2 * p + b
            pltpu.make_async_copy(
                x_hbm.at[pl.ds(0, RBLK)], bufs[b], sems[b]).wait()
            compute_block(g, b)

            @pl.when(g + 2 < NBLK)
            def _():
                dma_in(g + 2, b)
        return 0

    lax.fori_loop(0, NBLK // 2, pair_body, 0)
    pltpu.sync_copy(outb, out_hbm.at[pl.ds(row0, ROWS_PER_W)])


_sc_mesh = plsc.VectorSubcoreMesh(
    core_axis_name="c", subcore_axis_name="s", num_cores=NC, num_subcores=NS)

_sc_call = pl.kernel(
    _sc_body,
    out_type=jax.ShapeDtypeStruct((BATCH, NSEL), jnp.int32),
    mesh=_sc_mesh,
    scratch_types=[
        pltpu.VMEM((RBLK, EXP_DIM), jnp.float32),
        pltpu.VMEM((RBLK, EXP_DIM), jnp.float32),
        pltpu.VMEM((ROWS_PER_W, NSEL), jnp.int32),
        pltpu.SemaphoreType.DMA,
        pltpu.SemaphoreType.DMA,
    ],
    compiler_params=pltpu.CompilerParams(needs_layout_passes=False),
)


def kernel(sparse_code):
    return _sc_call(sparse_code)
